# baseline (speedup 1.0000x reference)
import functools

import jax
import jax.numpy as jnp
from jax import lax
from jax.experimental import pallas as pl
from jax.experimental.pallas import tpu as pltpu
from jax.experimental.pallas import tpu_sc as plsc

_N = 10000
_K = 32
_D = 128
_NPAD = 10240
_BLK = 256
_GRP = 8


def _linear_kernel(f_ref, w_ref, o_ref):
    o_ref[...] = lax.dot_general(
        f_ref[...], w_ref[...], (((1,), (0,)), ((), ())),
        preferred_element_type=jnp.float32)


def _attn_kernel(g_ref, vc_ref, vr_ref, b_ref, o_ref):
    G = g_ref[...]
    Vc = vc_ref[...]
    Vr = vr_ref[...]
    n_grp = _BLK // _GRP
    W_ = _GRP * _K
    bi = lax.broadcasted_iota(jnp.int32, (W_, W_), 0) // _K
    bj = lax.broadcasted_iota(jnp.int32, (W_, W_), 1) // _K
    blockmask = bi == bj
    eye = (lax.broadcasted_iota(jnp.int32, (W_, W_), 0)
           == lax.broadcasted_iota(jnp.int32, (W_, W_), 1)).astype(jnp.float32)
    dagg_rows = []
    for g in range(n_grp):
        X = G[g * W_:(g + 1) * W_, :]
        gram = lax.dot_general(
            X, X, (((1,), (1,)), ((), ())),
            preferred_element_type=jnp.float32)
        sq_c = jnp.sum(X * X, axis=1, keepdims=True)
        sq_r = lax.dot_general(
            sq_c, eye, (((0,), (0,)), ((), ())),
            precision=lax.Precision.HIGHEST,
            preferred_element_type=jnp.float32)
        v_c = Vc[g * W_:(g + 1) * W_] > 0
        v_r = Vr[g:g + 1, :] > 0
        d2 = jnp.maximum(sq_c + sq_r - 2.0 * gram, 0.0)
        dist = jnp.where(d2 > 0, jnp.sqrt(jnp.where(d2 > 0, d2, 1.0)), 0.0)
        dist = jnp.where(blockmask & v_c & v_r, dist, 0.0)
        dagg_rows.append(jnp.sum(dist, axis=0, keepdims=True))
    d_agg = jnp.concatenate(dagg_rows, axis=0)
    vmask = Vr > 0
    big = jnp.finfo(jnp.float32).max
    d_agg = jnp.where(vmask, d_agg, big)
    d_agg = jnp.where(jnp.isfinite(d_agg), d_agg, big)
    neg = -d_agg
    attn_segs = []
    for s in range(_GRP):
        seg = neg[:, s * _K:(s + 1) * _K]
        vseg = vmask[:, s * _K:(s + 1) * _K]
        m = jnp.max(seg, axis=1, keepdims=True)
        e = jnp.exp(seg - m)
        a = e / jnp.sum(e, axis=1, keepdims=True)
        a = a * vseg.astype(jnp.float32)
        a = a / jnp.sum(a, axis=1, keepdims=True)
        a = jnp.where(vseg, a, 0.0)
        attn_segs.append(a)
    attn = jnp.concatenate(attn_segs, axis=1)
    expand = (lax.broadcasted_iota(jnp.int32, (_GRP, W_), 1) // _K
              == lax.broadcasted_iota(jnp.int32, (_GRP, W_), 0)
              ).astype(jnp.float32)
    outs = []
    for g in range(n_grp):
        X = G[g * W_:(g + 1) * W_, :]
        a_mat = attn[g:g + 1, :] * expand
        outs.append(lax.dot_general(
            a_mat, X, (((1,), (0,)), ((), ())),
            preferred_element_type=jnp.float32))
    o_ref[...] = jnp.concatenate(outs, axis=0) + b_ref[...]


_NW = 32
_CH = 128


def _gather_rows(h, safe_flat):
    B = _NPAD * _K
    per_w = B // _NW
    n_ch = per_w // _CH
    mesh = plsc.VectorSubcoreMesh(core_axis_name="c", subcore_axis_name="s")

    @functools.partial(
        pl.kernel, mesh=mesh,
        out_type=jax.ShapeDtypeStruct((B, _D), jnp.float32),
        scratch_types=[pltpu.VMEM((_CH,), jnp.int32),
                       pltpu.VMEM((_CH, _D), jnp.float32),
                       pltpu.SemaphoreType.DMA],
    )
    def k(h_hbm, idx_hbm, out_hbm, idx_v, rows_v, sem):
        wid = lax.axis_index("s") * 2 + lax.axis_index("c")
        base = wid * per_w

        def body(i, carry):
            off = base + i * _CH
            pltpu.sync_copy(idx_hbm.at[pl.ds(off, _CH)], idx_v)
            pltpu.async_copy(h_hbm.at[idx_v], rows_v, sem).wait()
            pltpu.sync_copy(rows_v, out_hbm.at[pl.ds(off, _CH)])
            return carry

        lax.fori_loop(0, n_ch, body, 0)

    return k(h, safe_flat)


def _linear(feat, W):
    return pl.pallas_call(
        _linear_kernel,
        grid=(10,),
        in_specs=[pl.BlockSpec((1000, _D), lambda i: (i, 0)),
                  pl.BlockSpec((_D, _D), lambda i: (0, 0))],
        out_specs=pl.BlockSpec((1000, _D), lambda i: (i, 0)),
        out_shape=jax.ShapeDtypeStruct((_N, _D), jnp.float32),
    )(feat, W)


def _attention(gathered, vcol, vrow, b):
    nb = _NPAD // _BLK
    return pl.pallas_call(
        _attn_kernel,
        grid=(nb,),
        in_specs=[pl.BlockSpec((_BLK * _K, _D), lambda i: (i, 0)),
                  pl.BlockSpec((_BLK * _K, 1), lambda i: (i, 0)),
                  pl.BlockSpec((_BLK // _GRP, _GRP * _K), lambda i: (i, 0)),
                  pl.BlockSpec((1, _D), lambda i: (0, 0))],
        out_specs=pl.BlockSpec((_BLK, _D), lambda i: (i, 0)),
        out_shape=jax.ShapeDtypeStruct((_NPAD, _D), jnp.float32),
    )(gathered, vcol, vrow, b.reshape(1, _D))


def kernel(feat, edge_index, W, b):
    n = _N
    loops = jnp.arange(n, dtype=edge_index.dtype)
    rows = jnp.concatenate([edge_index[0], loops])
    cols = jnp.concatenate([edge_index[1], loops])
    h = _linear(feat, W)
    keys = rows * n + cols
    keys_s = jnp.sort(keys)
    rows_s = keys_s // n
    cols_s = keys_s % n
    deg = jnp.zeros((n,), jnp.int32).at[rows_s].add(1)
    row_start = jnp.cumsum(deg) - deg
    e = rows_s.shape[0]
    rank = jnp.arange(e, dtype=jnp.int32) - row_start[rows_s]
    topk = jnp.full((n, _K), -1, jnp.int32).at[rows_s, rank].set(
        cols_s, mode='drop')
    topk_p = jnp.concatenate(
        [topk, jnp.full((_NPAD - n, _K), -1, jnp.int32)], axis=0)
    safe = jnp.clip(topk_p, 0, n - 1)
    gathered = _gather_rows(h, safe.reshape(-1))
    valid = (topk_p != -1).astype(jnp.float32)
    vcol = valid.reshape(_NPAD * _K, 1)
    vrow = valid.reshape(_NPAD // _GRP, _GRP * _K)
    out = _attention(gathered, vcol, vrow, b)
    return out[:n]


# speedup vs baseline: 1.4271x; 1.4271x over previous
import functools

import jax
import jax.numpy as jnp
from jax import lax
from jax.experimental import pallas as pl
from jax.experimental.pallas import tpu as pltpu
from jax.experimental.pallas import tpu_sc as plsc

_N = 10000
_K = 32
_D = 128
_NPAD = 10240
_BLK = 256
_GRP = 8


def _linear_kernel(f_ref, w_ref, o_ref):
    o_ref[...] = lax.dot_general(
        f_ref[...], w_ref[...], (((1,), (0,)), ((), ())),
        preferred_element_type=jnp.float32)


def _attn_kernel(g_ref, vc_ref, vr_ref, b_ref, o_ref):
    G = g_ref[...]
    Vc = vc_ref[...]
    Vr = vr_ref[...]
    n_grp = _BLK // _GRP
    W_ = _GRP * _K
    bi = lax.broadcasted_iota(jnp.int32, (W_, W_), 0) // _K
    bj = lax.broadcasted_iota(jnp.int32, (W_, W_), 1) // _K
    blockmask = bi == bj
    eye = (lax.broadcasted_iota(jnp.int32, (W_, W_), 0)
           == lax.broadcasted_iota(jnp.int32, (W_, W_), 1)).astype(jnp.float32)
    dagg_rows = []
    for g in range(n_grp):
        X = G[g * W_:(g + 1) * W_, :]
        gram = lax.dot_general(
            X, X, (((1,), (1,)), ((), ())),
            preferred_element_type=jnp.float32)
        sq_c = jnp.sum(X * X, axis=1, keepdims=True)
        sq_r = lax.dot_general(
            sq_c, eye, (((0,), (0,)), ((), ())),
            precision=lax.Precision.HIGHEST,
            preferred_element_type=jnp.float32)
        v_c = Vc[g * W_:(g + 1) * W_] > 0
        v_r = Vr[g:g + 1, :] > 0
        d2 = jnp.maximum(sq_c + sq_r - 2.0 * gram, 0.0)
        dist = jnp.where(d2 > 0, jnp.sqrt(jnp.where(d2 > 0, d2, 1.0)), 0.0)
        dist = jnp.where(blockmask & v_c & v_r, dist, 0.0)
        dagg_rows.append(jnp.sum(dist, axis=0, keepdims=True))
    d_agg = jnp.concatenate(dagg_rows, axis=0)
    vmask = Vr > 0
    big = jnp.finfo(jnp.float32).max
    d_agg = jnp.where(vmask, d_agg, big)
    d_agg = jnp.where(jnp.isfinite(d_agg), d_agg, big)
    neg = -d_agg
    attn_segs = []
    for s in range(_GRP):
        seg = neg[:, s * _K:(s + 1) * _K]
        vseg = vmask[:, s * _K:(s + 1) * _K]
        m = jnp.max(seg, axis=1, keepdims=True)
        e = jnp.exp(seg - m)
        a = e / jnp.sum(e, axis=1, keepdims=True)
        a = a * vseg.astype(jnp.float32)
        a = a / jnp.sum(a, axis=1, keepdims=True)
        a = jnp.where(vseg, a, 0.0)
        attn_segs.append(a)
    attn = jnp.concatenate(attn_segs, axis=1)
    expand = (lax.broadcasted_iota(jnp.int32, (_GRP, W_), 1) // _K
              == lax.broadcasted_iota(jnp.int32, (_GRP, W_), 0)
              ).astype(jnp.float32)
    outs = []
    for g in range(n_grp):
        X = G[g * W_:(g + 1) * W_, :]
        a_mat = attn[g:g + 1, :] * expand
        outs.append(lax.dot_general(
            a_mat, X, (((1,), (0,)), ((), ())),
            preferred_element_type=jnp.float32))
    o_ref[...] = jnp.concatenate(outs, axis=0) + b_ref[...]


_NW = 32
_CH = 128


def _gather_rows(h, safe_flat):
    B = _NPAD * _K
    per_w = B // _NW
    n_ch = per_w // _CH
    mesh = plsc.VectorSubcoreMesh(core_axis_name="c", subcore_axis_name="s")

    @functools.partial(
        pl.kernel, mesh=mesh,
        out_type=jax.ShapeDtypeStruct((B, _D), jnp.float32),
        scratch_types=[pltpu.VMEM((_CH,), jnp.int32),
                       pltpu.VMEM((_CH, _D), jnp.float32),
                       pltpu.SemaphoreType.DMA],
    )
    def k(h_hbm, idx_hbm, out_hbm, idx_v, rows_v, sem):
        wid = lax.axis_index("s") * 2 + lax.axis_index("c")
        base = wid * per_w

        def body(i, carry):
            off = base + i * _CH
            pltpu.sync_copy(idx_hbm.at[pl.ds(off, _CH)], idx_v)
            pltpu.async_copy(h_hbm.at[idx_v], rows_v, sem).wait()
            pltpu.sync_copy(rows_v, out_hbm.at[pl.ds(off, _CH)])
            return carry

        lax.fori_loop(0, n_ch, body, 0)

    return k(h, safe_flat)


_EP = 331776
_SB = 2048
_RPW = _NPAD // _NW
_NB1 = 80


def _select_topk(rows_p, cols_p):
    mesh = plsc.VectorSubcoreMesh(core_axis_name="c", subcore_axis_name="s")

    @functools.partial(
        pl.kernel, mesh=mesh,
        compiler_params=pltpu.CompilerParams(needs_layout_passes=False),
        out_type=jax.ShapeDtypeStruct((_NPAD * _K,), jnp.int32),
        scratch_types=[
            pltpu.VMEM((_SB,), jnp.int32),
            pltpu.VMEM((_SB,), jnp.int32),
            pltpu.VMEM((_RPW * _NB1,), jnp.int32),
            pltpu.VMEM((_RPW * 128,), jnp.int32),
            pltpu.VMEM((_RPW * _K,), jnp.int32),
            pltpu.VMEM((_RPW,), jnp.int32),
            pltpu.VMEM((_RPW,), jnp.int32),
            pltpu.VMEM((_RPW,), jnp.int32),
            pltpu.VMEM((_RPW,), jnp.int32),
            pltpu.VMEM((_RPW,), jnp.int32),
            pltpu.VMEM((_RPW,), jnp.int32),
        ])
    def sel(rows_hbm, cols_hbm, out_hbm, rbuf, cbuf, h1, h2, tkl,
            bstar, cstar, need1, need2, sctr, ectr):
        wid = lax.axis_index("s") * 2 + lax.axis_index("c")
        base = wid * _RPW
        i16 = lax.broadcasted_iota(jnp.int32, (16,), 0)
        zeros16 = jnp.zeros((16,), jnp.int32)
        ones16 = jnp.ones((16,), jnp.int32)

        def zero_fill(ref, nwords):
            def zf(i, c):
                ref[pl.ds(i * 16, 16)] = zeros16
                return c
            lax.fori_loop(0, nwords // 16, zf, 0)

        zero_fill(h1, _RPW * _NB1)
        zero_fill(h2, _RPW * 128)
        zero_fill(sctr, _RPW)
        zero_fill(ectr, _RPW)

        def mf(i, c):
            tkl[pl.ds(i * 16, 16)] = zeros16 - 1
            return c
        lax.fori_loop(0, _RPW * _K // 16, mf, 0)

        def scan_edges(chunk_fn):
            def stage(s, c):
                pltpu.sync_copy(rows_hbm.at[pl.ds(s * _SB, _SB)], rbuf)
                pltpu.sync_copy(cols_hbm.at[pl.ds(s * _SB, _SB)], cbuf)

                def chunk(i, cc):
                    row = rbuf[pl.ds(i * 16, 16)]
                    col = cbuf[pl.ds(i * 16, 16)]
                    m = (row >= base) & (row < base + _RPW)
                    chunk_fn(row - base, col, m)
                    return cc
                lax.fori_loop(0, _SB // 16, chunk, 0)
                return c
            lax.fori_loop(0, _EP // _SB, stage, 0)

        def p1(r_loc, col, m):
            plsc.addupdate_scatter(h1, [r_loc * _NB1 + (col >> 7)], ones16,
                                   mask=m)
        scan_edges(p1)

        def l1(rg, c):
            r16 = rg * 16 + i16

            def bb(bkt, carry):
                cum, bst, below = carry
                v = plsc.load_gather(h1, [r16 * _NB1 + bkt])
                ncum = cum + v
                lt = (ncum < _K).astype(jnp.int32)
                return (ncum, bst + lt, below + v * lt)
            cum, bst, below = lax.fori_loop(
                0, _NB1, bb, (zeros16, zeros16, zeros16))
            plsc.store_scatter(bstar, [r16], bst)
            plsc.store_scatter(need1, [r16], _K - below)
            return c
        lax.fori_loop(0, _RPW // 16, l1, 0)

        def p2(r_loc, col, m):
            bst = plsc.load_gather(bstar, [r_loc], mask=m)
            m2 = m & ((col >> 7) == bst)
            plsc.addupdate_scatter(h2, [r_loc * 128 + (col & 127)], ones16,
                                   mask=m2)
        scan_edges(p2)

        def l2(rg, c):
            r16 = rg * 16 + i16
            n1 = plsc.load_gather(need1, [r16])

            def cb(cc, carry):
                cum, cst, below = carry
                v = plsc.load_gather(h2, [r16 * 128 + cc])
                ncum = cum + v
                lt = (ncum < n1).astype(jnp.int32)
                return (ncum, cst + lt, below + v * lt)
            cum, cst, below = lax.fori_loop(
                0, 128, cb, (zeros16, zeros16, zeros16))
            bst = plsc.load_gather(bstar, [r16])
            plsc.store_scatter(cstar, [r16], bst * 128 + cst)
            plsc.store_scatter(need2, [r16], n1 - below)
            return c
        lax.fori_loop(0, _RPW // 16, l2, 0)

        def p3(r_loc, col, m):
            cst = plsc.load_gather(cstar, [r_loc], mask=m)
            keep_lt = m & (col < cst)
            meq = m & (col == cst)
            oldeq = plsc.load_gather(ectr, [r_loc], mask=meq)
            occ_eq, _ = plsc.scan_count(r_loc, mask=meq)
            n2 = plsc.load_gather(need2, [r_loc], mask=meq)
            keep_eq = meq & ((oldeq + occ_eq - 1) < n2)
            plsc.addupdate_scatter(ectr, [r_loc], ones16, mask=meq)
            keep = keep_lt | keep_eq
            olds = plsc.load_gather(sctr, [r_loc], mask=keep)
            occ_s, _ = plsc.scan_count(r_loc, mask=keep)
            slot = olds + occ_s - 1
            keep = keep & (slot < _K)
            plsc.store_scatter(tkl, [r_loc * _K + slot], col, mask=keep)
            plsc.addupdate_scatter(sctr, [r_loc], ones16, mask=keep)
        scan_edges(p3)

        pltpu.sync_copy(tkl, out_hbm.at[pl.ds(wid * _RPW * _K, _RPW * _K)])

    return sel(rows_p, cols_p)


def _linear(feat, W):
    return pl.pallas_call(
        _linear_kernel,
        grid=(10,),
        in_specs=[pl.BlockSpec((1000, _D), lambda i: (i, 0)),
                  pl.BlockSpec((_D, _D), lambda i: (0, 0))],
        out_specs=pl.BlockSpec((1000, _D), lambda i: (i, 0)),
        out_shape=jax.ShapeDtypeStruct((_N, _D), jnp.float32),
    )(feat, W)


def _attention(gathered, vcol, vrow, b):
    nb = _NPAD // _BLK
    return pl.pallas_call(
        _attn_kernel,
        grid=(nb,),
        in_specs=[pl.BlockSpec((_BLK * _K, _D), lambda i: (i, 0)),
                  pl.BlockSpec((_BLK * _K, 1), lambda i: (i, 0)),
                  pl.BlockSpec((_BLK // _GRP, _GRP * _K), lambda i: (i, 0)),
                  pl.BlockSpec((1, _D), lambda i: (0, 0))],
        out_specs=pl.BlockSpec((_BLK, _D), lambda i: (i, 0)),
        out_shape=jax.ShapeDtypeStruct((_NPAD, _D), jnp.float32),
    )(gathered, vcol, vrow, b.reshape(1, _D))


def kernel(feat, edge_index, W, b):
    n = _N
    e = edge_index.shape[1]
    loops = jnp.arange(n, dtype=edge_index.dtype)
    pad = _EP - e - n
    rows_p = jnp.concatenate(
        [edge_index[0], loops, jnp.full((pad,), _NPAD - 1, jnp.int32)])
    cols_p = jnp.concatenate(
        [edge_index[1], loops, jnp.zeros((pad,), jnp.int32)])
    h = _linear(feat, W)
    topk_p = _select_topk(rows_p, cols_p).reshape(_NPAD, _K)
    safe = jnp.clip(topk_p, 0, n - 1)
    gathered = _gather_rows(h, safe.reshape(-1))
    valid = (topk_p != -1).astype(jnp.float32)
    vcol = valid.reshape(_NPAD * _K, 1)
    vrow = valid.reshape(_NPAD // _GRP, _GRP * _K)
    out = _attention(gathered, vcol, vrow, b)
    return out[:n]


# speedup vs baseline: 1.4354x; 1.0058x over previous
import functools

import jax
import jax.numpy as jnp
from jax import lax
from jax.experimental import pallas as pl
from jax.experimental.pallas import tpu as pltpu
from jax.experimental.pallas import tpu_sc as plsc

_N = 10000
_K = 32
_D = 128
_NPAD = 10240
_BLK = 256
_GRP = 8


def _linear_kernel(f_ref, w_ref, o_ref):
    o_ref[...] = lax.dot_general(
        f_ref[...], w_ref[...], (((1,), (0,)), ((), ())),
        preferred_element_type=jnp.float32)


def _attn_kernel(g_ref, vc_ref, vr_ref, b_ref, o_ref):
    G = g_ref[...]
    Vc = vc_ref[...]
    Vr = vr_ref[...]
    n_grp = _BLK // _GRP
    W_ = _GRP * _K
    bi = lax.broadcasted_iota(jnp.int32, (W_, W_), 0) // _K
    bj = lax.broadcasted_iota(jnp.int32, (W_, W_), 1) // _K
    blockmask = bi == bj
    eye = (lax.broadcasted_iota(jnp.int32, (W_, W_), 0)
           == lax.broadcasted_iota(jnp.int32, (W_, W_), 1)).astype(jnp.float32)
    dagg_rows = []
    for g in range(n_grp):
        X = G[g * W_:(g + 1) * W_, :]
        gram = lax.dot_general(
            X, X, (((1,), (1,)), ((), ())),
            preferred_element_type=jnp.float32)
        sq_c = jnp.sum(X * X, axis=1, keepdims=True)
        sq_r = lax.dot_general(
            sq_c, eye, (((0,), (0,)), ((), ())),
            precision=lax.Precision.HIGHEST,
            preferred_element_type=jnp.float32)
        v_c = Vc[g * W_:(g + 1) * W_] > 0
        v_r = Vr[g:g + 1, :] > 0
        d2 = jnp.maximum(sq_c + sq_r - 2.0 * gram, 0.0)
        dist = jnp.where(d2 > 0, jnp.sqrt(jnp.where(d2 > 0, d2, 1.0)), 0.0)
        dist = jnp.where(blockmask & v_c & v_r, dist, 0.0)
        dagg_rows.append(jnp.sum(dist, axis=0, keepdims=True))
    d_agg = jnp.concatenate(dagg_rows, axis=0)
    vmask = Vr > 0
    big = jnp.finfo(jnp.float32).max
    d_agg = jnp.where(vmask, d_agg, big)
    d_agg = jnp.where(jnp.isfinite(d_agg), d_agg, big)
    neg = -d_agg
    attn_segs = []
    for s in range(_GRP):
        seg = neg[:, s * _K:(s + 1) * _K]
        vseg = vmask[:, s * _K:(s + 1) * _K]
        m = jnp.max(seg, axis=1, keepdims=True)
        e = jnp.exp(seg - m)
        a = e / jnp.sum(e, axis=1, keepdims=True)
        a = a * vseg.astype(jnp.float32)
        a = a / jnp.sum(a, axis=1, keepdims=True)
        a = jnp.where(vseg, a, 0.0)
        attn_segs.append(a)
    attn = jnp.concatenate(attn_segs, axis=1)
    expand = (lax.broadcasted_iota(jnp.int32, (_GRP, W_), 1) // _K
              == lax.broadcasted_iota(jnp.int32, (_GRP, W_), 0)
              ).astype(jnp.float32)
    outs = []
    for g in range(n_grp):
        X = G[g * W_:(g + 1) * W_, :]
        a_mat = attn[g:g + 1, :] * expand
        outs.append(lax.dot_general(
            a_mat, X, (((1,), (0,)), ((), ())),
            preferred_element_type=jnp.float32))
    o_ref[...] = jnp.concatenate(outs, axis=0) + b_ref[...]


_NW = 32
_CH = 128


def _gather_rows(h, safe_flat):
    B = _NPAD * _K
    per_w = B // _NW
    n_ch = per_w // _CH
    mesh = plsc.VectorSubcoreMesh(core_axis_name="c", subcore_axis_name="s")

    nbuf = 4
    per_body = 8
    n_body = n_ch // per_body

    @functools.partial(
        pl.kernel, mesh=mesh,
        out_type=jax.ShapeDtypeStruct((B, _D), jnp.float32),
        scratch_types=[pltpu.VMEM((nbuf, _CH), jnp.int32),
                       pltpu.VMEM((nbuf, _CH, _D), jnp.float32),
                       pltpu.SemaphoreType.DMA((nbuf,)),
                       pltpu.SemaphoreType.DMA((nbuf,))],
    )
    def k(h_hbm, idx_hbm, out_hbm, idx_v, rows_v, gsems, osems):
        wid = lax.axis_index("s") * 2 + lax.axis_index("c")
        base = wid * per_w

        def body(t, carry):
            base_c = base + t * per_body * _CH
            gat = [None] * nbuf
            outs = [None] * nbuf
            for kk in range(per_body + nbuf - 1):
                bb = kk % nbuf
                if kk < per_body:
                    if kk >= nbuf:
                        outs[bb].wait()
                    off = base_c + kk * _CH
                    pltpu.sync_copy(idx_hbm.at[pl.ds(off, _CH)],
                                    idx_v.at[bb])
                    gat[bb] = pltpu.async_copy(
                        h_hbm.at[idx_v.at[bb]], rows_v.at[bb], gsems.at[bb])
                j = kk - (nbuf - 1)
                if j >= 0:
                    bj = j % nbuf
                    gat[bj].wait()
                    offj = base_c + j * _CH
                    outs[bj] = pltpu.async_copy(
                        rows_v.at[bj], out_hbm.at[pl.ds(offj, _CH)],
                        osems.at[bj])
            for bb in range(nbuf):
                outs[bb].wait()
            return carry

        lax.fori_loop(0, n_body, body, 0)

    return k(h, safe_flat)


_EP = 331776
_SB = 2048
_RPW = _NPAD // _NW
_NB1 = 80


def _select_topk(rows_p, cols_p):
    mesh = plsc.VectorSubcoreMesh(core_axis_name="c", subcore_axis_name="s")

    @functools.partial(
        pl.kernel, mesh=mesh,
        compiler_params=pltpu.CompilerParams(needs_layout_passes=False),
        out_type=jax.ShapeDtypeStruct((_NPAD * _K,), jnp.int32),
        scratch_types=[
            pltpu.VMEM((_SB,), jnp.int32),
            pltpu.VMEM((_SB,), jnp.int32),
            pltpu.VMEM((_RPW * _NB1,), jnp.int32),
            pltpu.VMEM((_RPW * 128,), jnp.int32),
            pltpu.VMEM((_RPW * _K,), jnp.int32),
            pltpu.VMEM((_RPW,), jnp.int32),
            pltpu.VMEM((_RPW,), jnp.int32),
            pltpu.VMEM((_RPW,), jnp.int32),
            pltpu.VMEM((_RPW,), jnp.int32),
            pltpu.VMEM((_RPW,), jnp.int32),
            pltpu.VMEM((_RPW,), jnp.int32),
        ])
    def sel(rows_hbm, cols_hbm, out_hbm, rbuf, cbuf, h1, h2, tkl,
            bstar, cstar, need1, need2, sctr, ectr):
        wid = lax.axis_index("s") * 2 + lax.axis_index("c")
        base = wid * _RPW
        i16 = lax.broadcasted_iota(jnp.int32, (16,), 0)
        zeros16 = jnp.zeros((16,), jnp.int32)
        ones16 = jnp.ones((16,), jnp.int32)

        def zero_fill(ref, nwords):
            def zf(i, c):
                ref[pl.ds(i * 16, 16)] = zeros16
                return c
            lax.fori_loop(0, nwords // 16, zf, 0)

        zero_fill(h1, _RPW * _NB1)
        zero_fill(h2, _RPW * 128)
        zero_fill(sctr, _RPW)
        zero_fill(ectr, _RPW)

        def mf(i, c):
            tkl[pl.ds(i * 16, 16)] = zeros16 - 1
            return c
        lax.fori_loop(0, _RPW * _K // 16, mf, 0)

        def scan_edges(chunk_fn):
            def stage(s, c):
                pltpu.sync_copy(rows_hbm.at[pl.ds(s * _SB, _SB)], rbuf)
                pltpu.sync_copy(cols_hbm.at[pl.ds(s * _SB, _SB)], cbuf)

                def chunk(i, cc):
                    row = rbuf[pl.ds(i * 16, 16)]
                    col = cbuf[pl.ds(i * 16, 16)]
                    m = (row >= base) & (row < base + _RPW)
                    chunk_fn(row - base, col, m)
                    return cc
                lax.fori_loop(0, _SB // 16, chunk, 0)
                return c
            lax.fori_loop(0, _EP // _SB, stage, 0)

        def p1(r_loc, col, m):
            plsc.addupdate_scatter(h1, [r_loc * _NB1 + (col >> 7)], ones16,
                                   mask=m)
        scan_edges(p1)

        def l1(rg, c):
            r16 = rg * 16 + i16

            def bb(bkt, carry):
                cum, bst, below = carry
                v = plsc.load_gather(h1, [r16 * _NB1 + bkt])
                ncum = cum + v
                lt = (ncum < _K).astype(jnp.int32)
                return (ncum, bst + lt, below + v * lt)
            cum, bst, below = lax.fori_loop(
                0, _NB1, bb, (zeros16, zeros16, zeros16))
            plsc.store_scatter(bstar, [r16], bst)
            plsc.store_scatter(need1, [r16], _K - below)
            return c
        lax.fori_loop(0, _RPW // 16, l1, 0)

        def p2(r_loc, col, m):
            bst = plsc.load_gather(bstar, [r_loc], mask=m)
            m2 = m & ((col >> 7) == bst)
            plsc.addupdate_scatter(h2, [r_loc * 128 + (col & 127)], ones16,
                                   mask=m2)
        scan_edges(p2)

        def l2(rg, c):
            r16 = rg * 16 + i16
            n1 = plsc.load_gather(need1, [r16])

            def cb(cc, carry):
                cum, cst, below = carry
                v = plsc.load_gather(h2, [r16 * 128 + cc])
                ncum = cum + v
                lt = (ncum < n1).astype(jnp.int32)
                return (ncum, cst + lt, below + v * lt)
            cum, cst, below = lax.fori_loop(
                0, 128, cb, (zeros16, zeros16, zeros16))
            bst = plsc.load_gather(bstar, [r16])
            plsc.store_scatter(cstar, [r16], bst * 128 + cst)
            plsc.store_scatter(need2, [r16], n1 - below)
            return c
        lax.fori_loop(0, _RPW // 16, l2, 0)

        def p3(r_loc, col, m):
            cst = plsc.load_gather(cstar, [r_loc], mask=m)
            keep_lt = m & (col < cst)
            meq = m & (col == cst)
            oldeq = plsc.load_gather(ectr, [r_loc], mask=meq)
            occ_eq, _ = plsc.scan_count(r_loc, mask=meq)
            n2 = plsc.load_gather(need2, [r_loc], mask=meq)
            keep_eq = meq & ((oldeq + occ_eq - 1) < n2)
            plsc.addupdate_scatter(ectr, [r_loc], ones16, mask=meq)
            keep = keep_lt | keep_eq
            olds = plsc.load_gather(sctr, [r_loc], mask=keep)
            occ_s, _ = plsc.scan_count(r_loc, mask=keep)
            slot = olds + occ_s - 1
            keep = keep & (slot < _K)
            plsc.store_scatter(tkl, [r_loc * _K + slot], col, mask=keep)
            plsc.addupdate_scatter(sctr, [r_loc], ones16, mask=keep)
        scan_edges(p3)

        pltpu.sync_copy(tkl, out_hbm.at[pl.ds(wid * _RPW * _K, _RPW * _K)])

    return sel(rows_p, cols_p)


def _linear(feat, W):
    return pl.pallas_call(
        _linear_kernel,
        grid=(10,),
        in_specs=[pl.BlockSpec((1000, _D), lambda i: (i, 0)),
                  pl.BlockSpec((_D, _D), lambda i: (0, 0))],
        out_specs=pl.BlockSpec((1000, _D), lambda i: (i, 0)),
        out_shape=jax.ShapeDtypeStruct((_N, _D), jnp.float32),
    )(feat, W)


def _attention(gathered, vcol, vrow, b):
    nb = _NPAD // _BLK
    return pl.pallas_call(
        _attn_kernel,
        grid=(nb,),
        in_specs=[pl.BlockSpec((_BLK * _K, _D), lambda i: (i, 0)),
                  pl.BlockSpec((_BLK * _K, 1), lambda i: (i, 0)),
                  pl.BlockSpec((_BLK // _GRP, _GRP * _K), lambda i: (i, 0)),
                  pl.BlockSpec((1, _D), lambda i: (0, 0))],
        out_specs=pl.BlockSpec((_BLK, _D), lambda i: (i, 0)),
        out_shape=jax.ShapeDtypeStruct((_NPAD, _D), jnp.float32),
    )(gathered, vcol, vrow, b.reshape(1, _D))


def kernel(feat, edge_index, W, b):
    n = _N
    e = edge_index.shape[1]
    loops = jnp.arange(n, dtype=edge_index.dtype)
    pad = _EP - e - n
    rows_p = jnp.concatenate(
        [edge_index[0], loops, jnp.full((pad,), _NPAD - 1, jnp.int32)])
    cols_p = jnp.concatenate(
        [edge_index[1], loops, jnp.zeros((pad,), jnp.int32)])
    h = _linear(feat, W)
    topk_p = _select_topk(rows_p, cols_p).reshape(_NPAD, _K)
    safe = jnp.clip(topk_p, 0, n - 1)
    gathered = _gather_rows(h, safe.reshape(-1))
    valid = (topk_p != -1).astype(jnp.float32)
    vcol = valid.reshape(_NPAD * _K, 1)
    vrow = valid.reshape(_NPAD // _GRP, _GRP * _K)
    out = _attention(gathered, vcol, vrow, b)
    return out[:n]


# speedup vs baseline: 1.7116x; 1.1924x over previous
import functools

import jax
import jax.numpy as jnp
from jax import lax
from jax.experimental import pallas as pl
from jax.experimental.pallas import tpu as pltpu
from jax.experimental.pallas import tpu_sc as plsc

_N = 10000
_K = 32
_D = 128
_NPAD = 10240
_BLK = 256
_GRP = 8


def _linear_kernel(f_ref, w_ref, o_ref):
    o_ref[...] = lax.dot_general(
        f_ref[...], w_ref[...], (((1,), (0,)), ((), ())),
        preferred_element_type=jnp.float32)


def _attn_kernel(g_ref, vc_ref, vr_ref, b_ref, o_ref):
    G = g_ref[...]
    Vc = vc_ref[...]
    Vr = vr_ref[...]
    n_grp = _BLK // _GRP
    W_ = _GRP * _K
    bi = lax.broadcasted_iota(jnp.int32, (W_, W_), 0) // _K
    bj = lax.broadcasted_iota(jnp.int32, (W_, W_), 1) // _K
    blockmask = bi == bj
    eye = (lax.broadcasted_iota(jnp.int32, (W_, W_), 0)
           == lax.broadcasted_iota(jnp.int32, (W_, W_), 1)).astype(jnp.float32)
    dagg_rows = []
    for g in range(n_grp):
        X = G[g * W_:(g + 1) * W_, :]
        gram = lax.dot_general(
            X, X, (((1,), (1,)), ((), ())),
            preferred_element_type=jnp.float32)
        sq_c = jnp.sum(X * X, axis=1, keepdims=True)
        sq_r = lax.dot_general(
            sq_c, eye, (((0,), (0,)), ((), ())),
            precision=lax.Precision.HIGHEST,
            preferred_element_type=jnp.float32)
        v_c = Vc[g * W_:(g + 1) * W_] > 0
        v_r = Vr[g:g + 1, :] > 0
        d2 = jnp.maximum(sq_c + sq_r - 2.0 * gram, 0.0)
        dist = jnp.where(d2 > 0, jnp.sqrt(jnp.where(d2 > 0, d2, 1.0)), 0.0)
        dist = jnp.where(blockmask & v_c & v_r, dist, 0.0)
        dagg_rows.append(jnp.sum(dist, axis=0, keepdims=True))
    d_agg = jnp.concatenate(dagg_rows, axis=0)
    vmask = Vr > 0
    big = jnp.finfo(jnp.float32).max
    d_agg = jnp.where(vmask, d_agg, big)
    d_agg = jnp.where(jnp.isfinite(d_agg), d_agg, big)
    neg = -d_agg
    attn_segs = []
    for s in range(_GRP):
        seg = neg[:, s * _K:(s + 1) * _K]
        vseg = vmask[:, s * _K:(s + 1) * _K]
        m = jnp.max(seg, axis=1, keepdims=True)
        e = jnp.exp(seg - m)
        a = e / jnp.sum(e, axis=1, keepdims=True)
        a = a * vseg.astype(jnp.float32)
        a = a / jnp.sum(a, axis=1, keepdims=True)
        a = jnp.where(vseg, a, 0.0)
        attn_segs.append(a)
    attn = jnp.concatenate(attn_segs, axis=1)
    expand = (lax.broadcasted_iota(jnp.int32, (_GRP, W_), 1) // _K
              == lax.broadcasted_iota(jnp.int32, (_GRP, W_), 0)
              ).astype(jnp.float32)
    outs = []
    for g in range(n_grp):
        X = G[g * W_:(g + 1) * W_, :]
        a_mat = attn[g:g + 1, :] * expand
        outs.append(lax.dot_general(
            a_mat, X, (((1,), (0,)), ((), ())),
            preferred_element_type=jnp.float32))
    o_ref[...] = jnp.concatenate(outs, axis=0) + b_ref[...]


_NW = 32
_CH = 128


def _gather_rows(h, safe_flat):
    B = safe_flat.shape[0]
    per_w = B // _NW
    n_ch = per_w // _CH
    mesh = plsc.VectorSubcoreMesh(core_axis_name="c", subcore_axis_name="s")

    nbuf = 4
    per_body = 8
    n_body = n_ch // per_body

    @functools.partial(
        pl.kernel, mesh=mesh,
        out_type=jax.ShapeDtypeStruct((B, _D), jnp.float32),
        scratch_types=[pltpu.VMEM((nbuf, _CH), jnp.int32),
                       pltpu.VMEM((nbuf, _CH, _D), jnp.float32),
                       pltpu.SemaphoreType.DMA((nbuf,)),
                       pltpu.SemaphoreType.DMA((nbuf,))],
    )
    def k(h_hbm, idx_hbm, out_hbm, idx_v, rows_v, gsems, osems):
        wid = lax.axis_index("s") * 2 + lax.axis_index("c")
        base = wid * per_w

        def body(t, carry):
            base_c = base + t * per_body * _CH
            gat = [None] * nbuf
            outs = [None] * nbuf
            for kk in range(per_body + nbuf - 1):
                bb = kk % nbuf
                if kk < per_body:
                    if kk >= nbuf:
                        outs[bb].wait()
                    off = base_c + kk * _CH
                    pltpu.sync_copy(idx_hbm.at[pl.ds(off, _CH)],
                                    idx_v.at[bb])
                    gat[bb] = pltpu.async_copy(
                        h_hbm.at[idx_v.at[bb]], rows_v.at[bb], gsems.at[bb])
                j = kk - (nbuf - 1)
                if j >= 0:
                    bj = j % nbuf
                    gat[bj].wait()
                    offj = base_c + j * _CH
                    outs[bj] = pltpu.async_copy(
                        rows_v.at[bj], out_hbm.at[pl.ds(offj, _CH)],
                        osems.at[bj])
            for bb in range(nbuf):
                outs[bb].wait()
            return carry

        lax.fori_loop(0, n_body, body, 0)

    return k(h, safe_flat)


_EP = 331776
_SB = 2048
_RPW = _NPAD // _NW
_NB1 = 80


def _select_topk(rows_p, cols_p):
    mesh = plsc.VectorSubcoreMesh(core_axis_name="c", subcore_axis_name="s")

    @functools.partial(
        pl.kernel, mesh=mesh,
        compiler_params=pltpu.CompilerParams(needs_layout_passes=False),
        out_type=jax.ShapeDtypeStruct((_NPAD * _K,), jnp.int32),
        scratch_types=[
            pltpu.VMEM((2, _SB), jnp.int32),
            pltpu.VMEM((2, _SB), jnp.int32),
            pltpu.SemaphoreType.DMA((2,)),
            pltpu.SemaphoreType.DMA((2,)),
            pltpu.VMEM((_RPW * _NB1,), jnp.int32),
            pltpu.VMEM((_RPW * 128,), jnp.int32),
            pltpu.VMEM((_RPW * _K,), jnp.int32),
            pltpu.VMEM((_RPW,), jnp.int32),
            pltpu.VMEM((_RPW,), jnp.int32),
            pltpu.VMEM((_RPW,), jnp.int32),
            pltpu.VMEM((_RPW,), jnp.int32),
            pltpu.VMEM((_RPW,), jnp.int32),
            pltpu.VMEM((_RPW,), jnp.int32),
        ])
    def sel(rows_hbm, cols_hbm, out_hbm, rbuf, cbuf, ssr, ssc, h1, h2, tkl,
            bstar, cstar, need1, need2, sctr, ectr):
        wid = lax.axis_index("s") * 2 + lax.axis_index("c")
        base = wid * _RPW
        i16 = lax.broadcasted_iota(jnp.int32, (16,), 0)
        zeros16 = jnp.zeros((16,), jnp.int32)
        ones16 = jnp.ones((16,), jnp.int32)

        def zero_fill(ref, nwords):
            def zf(i, c):
                ref[pl.ds(i * 16, 16)] = zeros16
                return c
            lax.fori_loop(0, nwords // 16, zf, 0)

        zero_fill(h1, _RPW * _NB1)
        zero_fill(h2, _RPW * 128)
        zero_fill(sctr, _RPW)
        zero_fill(ectr, _RPW)

        def mf(i, c):
            tkl[pl.ds(i * 16, 16)] = zeros16 - 1
            return c
        lax.fori_loop(0, _RPW * _K // 16, mf, 0)

        nstg = _EP // _SB

        def scan_edges(chunk_fn):
            def start_buf(bb, s):
                pltpu.async_copy(rows_hbm.at[pl.ds(s * _SB, _SB)],
                                 rbuf.at[bb], ssr.at[bb])
                pltpu.async_copy(cols_hbm.at[pl.ds(s * _SB, _SB)],
                                 cbuf.at[bb], ssc.at[bb])

            def wait_buf(bb):
                pltpu.make_async_copy(rows_hbm.at[pl.ds(0, _SB)],
                                      rbuf.at[bb], ssr.at[bb]).wait()
                pltpu.make_async_copy(cols_hbm.at[pl.ds(0, _SB)],
                                      cbuf.at[bb], ssc.at[bb]).wait()

            def process(bb):
                def chunk(i, cc):
                    row = rbuf[bb, pl.ds(i * 16, 16)]
                    col = cbuf[bb, pl.ds(i * 16, 16)]
                    m = (row >= base) & (row < base + _RPW)
                    chunk_fn(row - base, col, m)
                    return cc
                lax.fori_loop(0, _SB // 16, chunk, 0)

            start_buf(0, 0)

            def stage2(t, c):
                s0 = t * 2
                start_buf(1, s0 + 1)
                wait_buf(0)
                process(0)

                @pl.when(s0 + 2 < nstg)
                def _():
                    start_buf(0, s0 + 2)
                wait_buf(1)
                process(1)
                return c
            lax.fori_loop(0, nstg // 2, stage2, 0)

        def p1(r_loc, col, m):
            plsc.addupdate_scatter(h1, [r_loc * _NB1 + (col >> 7)], ones16,
                                   mask=m)
        scan_edges(p1)

        def l1(rg, c):
            r16 = rg * 16 + i16

            def bb(bkt, carry):
                cum, bst, below = carry
                v = plsc.load_gather(h1, [r16 * _NB1 + bkt])
                ncum = cum + v
                lt = (ncum < _K).astype(jnp.int32)
                return (ncum, bst + lt, below + v * lt)
            cum, bst, below = lax.fori_loop(
                0, _NB1, bb, (zeros16, zeros16, zeros16))
            plsc.store_scatter(bstar, [r16], bst)
            plsc.store_scatter(need1, [r16], _K - below)
            return c
        lax.fori_loop(0, _RPW // 16, l1, 0)

        def p2(r_loc, col, m):
            bst = plsc.load_gather(bstar, [r_loc], mask=m)
            m2 = m & ((col >> 7) == bst)
            plsc.addupdate_scatter(h2, [r_loc * 128 + (col & 127)], ones16,
                                   mask=m2)
        scan_edges(p2)

        def l2(rg, c):
            r16 = rg * 16 + i16
            n1 = plsc.load_gather(need1, [r16])

            def cb(cc, carry):
                cum, cst, below = carry
                v = plsc.load_gather(h2, [r16 * 128 + cc])
                ncum = cum + v
                lt = (ncum < n1).astype(jnp.int32)
                return (ncum, cst + lt, below + v * lt)
            cum, cst, below = lax.fori_loop(
                0, 128, cb, (zeros16, zeros16, zeros16))
            bst = plsc.load_gather(bstar, [r16])
            plsc.store_scatter(cstar, [r16], bst * 128 + cst)
            plsc.store_scatter(need2, [r16], n1 - below)
            return c
        lax.fori_loop(0, _RPW // 16, l2, 0)

        def p3(r_loc, col, m):
            cst = plsc.load_gather(cstar, [r_loc], mask=m)
            keep_lt = m & (col < cst)
            meq = m & (col == cst)
            oldeq = plsc.load_gather(ectr, [r_loc], mask=meq)
            occ_eq, _ = plsc.scan_count(r_loc, mask=meq)
            n2 = plsc.load_gather(need2, [r_loc], mask=meq)
            keep_eq = meq & ((oldeq + occ_eq - 1) < n2)
            plsc.addupdate_scatter(ectr, [r_loc], ones16, mask=meq)
            keep = keep_lt | keep_eq
            olds = plsc.load_gather(sctr, [r_loc], mask=keep)
            occ_s, _ = plsc.scan_count(r_loc, mask=keep)
            slot = olds + occ_s - 1
            keep = keep & (slot < _K)
            plsc.store_scatter(tkl, [r_loc * _K + slot], col, mask=keep)
            plsc.addupdate_scatter(sctr, [r_loc], ones16, mask=keep)
        scan_edges(p3)

        pltpu.sync_copy(tkl, out_hbm.at[pl.ds(wid * _RPW * _K, _RPW * _K)])

    return sel(rows_p, cols_p)


def _linear(feat, W):
    return pl.pallas_call(
        _linear_kernel,
        grid=(10,),
        in_specs=[pl.BlockSpec((1000, _D), lambda i: (i, 0)),
                  pl.BlockSpec((_D, _D), lambda i: (0, 0))],
        out_specs=pl.BlockSpec((1000, _D), lambda i: (i, 0)),
        out_shape=jax.ShapeDtypeStruct((_N, _D), jnp.float32),
    )(feat, W)


def _attention(gathered, vcol, vrow, b):
    npart = vcol.shape[0] // _K
    nb = npart // _BLK
    return pl.pallas_call(
        _attn_kernel,
        grid=(nb,),
        in_specs=[pl.BlockSpec((_BLK * _K, _D), lambda i: (i, 0)),
                  pl.BlockSpec((_BLK * _K, 1), lambda i: (i, 0)),
                  pl.BlockSpec((_BLK // _GRP, _GRP * _K), lambda i: (i, 0)),
                  pl.BlockSpec((1, _D), lambda i: (0, 0))],
        out_specs=pl.BlockSpec((_BLK, _D), lambda i: (i, 0)),
        out_shape=jax.ShapeDtypeStruct((npart, _D), jnp.float32),
    )(gathered, vcol, vrow, b.reshape(1, _D))


def kernel(feat, edge_index, W, b):
    n = _N
    e = edge_index.shape[1]
    loops = jnp.arange(n, dtype=edge_index.dtype)
    pad = _EP - e - n
    rows_p = jnp.concatenate(
        [edge_index[0], loops, jnp.full((pad,), _NPAD - 1, jnp.int32)])
    cols_p = jnp.concatenate(
        [edge_index[1], loops, jnp.zeros((pad,), jnp.int32)])
    h = _linear(feat, W)
    topk_p = _select_topk(rows_p, cols_p).reshape(_NPAD, _K)
    safe = jnp.clip(topk_p, 0, n - 1)
    gathered = _gather_rows(h, safe.reshape(-1))
    valid = (topk_p != -1).astype(jnp.float32)
    vcol = valid.reshape(_NPAD * _K, 1)
    vrow = valid.reshape(_NPAD // _GRP, _GRP * _K)
    out = _attention(gathered, vcol, vrow, b)
    return out[:n]


# speedup vs baseline: 1.7765x; 1.0379x over previous
import functools

import jax
import jax.numpy as jnp
from jax import lax
from jax.experimental import pallas as pl
from jax.experimental.pallas import tpu as pltpu
from jax.experimental.pallas import tpu_sc as plsc

_N = 10000
_K = 32
_D = 128
_NPAD = 10240
_BLK = 256
_GRP = 8


def _linear_kernel(f_ref, w_ref, o_ref):
    o_ref[...] = lax.dot_general(
        f_ref[...], w_ref[...], (((1,), (0,)), ((), ())),
        preferred_element_type=jnp.float32)


def _attn_kernel(g_ref, vc_ref, vr_ref, b_ref, o_ref):
    G = g_ref[...]
    Vc = vc_ref[...]
    Vr = vr_ref[...]
    n_grp = _BLK // _GRP
    W_ = _GRP * _K
    bi = lax.broadcasted_iota(jnp.int32, (W_, W_), 0) // _K
    bj = lax.broadcasted_iota(jnp.int32, (W_, W_), 1) // _K
    blockmask = bi == bj
    eye = (lax.broadcasted_iota(jnp.int32, (W_, W_), 0)
           == lax.broadcasted_iota(jnp.int32, (W_, W_), 1)).astype(jnp.float32)
    dagg_rows = []
    for g in range(n_grp):
        X = G[g * W_:(g + 1) * W_, :]
        gram = lax.dot_general(
            X, X, (((1,), (1,)), ((), ())),
            preferred_element_type=jnp.float32)
        sq_c = jnp.sum(X * X, axis=1, keepdims=True)
        sq_r = lax.dot_general(
            sq_c, eye, (((0,), (0,)), ((), ())),
            precision=lax.Precision.HIGHEST,
            preferred_element_type=jnp.float32)
        v_c = Vc[g * W_:(g + 1) * W_] > 0
        v_r = Vr[g:g + 1, :] > 0
        d2 = jnp.maximum(sq_c + sq_r - 2.0 * gram, 0.0)
        dist = jnp.where(d2 > 0, jnp.sqrt(jnp.where(d2 > 0, d2, 1.0)), 0.0)
        dist = jnp.where(blockmask & v_c & v_r, dist, 0.0)
        dagg_rows.append(jnp.sum(dist, axis=0, keepdims=True))
    d_agg = jnp.concatenate(dagg_rows, axis=0)
    vmask = Vr > 0
    big = jnp.finfo(jnp.float32).max
    d_agg = jnp.where(vmask, d_agg, big)
    d_agg = jnp.where(jnp.isfinite(d_agg), d_agg, big)
    neg = -d_agg
    attn_segs = []
    for s in range(_GRP):
        seg = neg[:, s * _K:(s + 1) * _K]
        vseg = vmask[:, s * _K:(s + 1) * _K]
        m = jnp.max(seg, axis=1, keepdims=True)
        e = jnp.exp(seg - m)
        a = e / jnp.sum(e, axis=1, keepdims=True)
        a = a * vseg.astype(jnp.float32)
        a = a / jnp.sum(a, axis=1, keepdims=True)
        a = jnp.where(vseg, a, 0.0)
        attn_segs.append(a)
    attn = jnp.concatenate(attn_segs, axis=1)
    expand = (lax.broadcasted_iota(jnp.int32, (_GRP, W_), 1) // _K
              == lax.broadcasted_iota(jnp.int32, (_GRP, W_), 0)
              ).astype(jnp.float32)
    outs = []
    for g in range(n_grp):
        X = G[g * W_:(g + 1) * W_, :]
        a_mat = attn[g:g + 1, :] * expand
        outs.append(lax.dot_general(
            a_mat, X, (((1,), (0,)), ((), ())),
            preferred_element_type=jnp.float32))
    o_ref[...] = jnp.concatenate(outs, axis=0) + b_ref[...]


_NW = 32
_CH = 128


def _gather_rows(h, safe_flat):
    B = safe_flat.shape[0]
    per_w = B // _NW
    n_ch = per_w // _CH
    mesh = plsc.VectorSubcoreMesh(core_axis_name="c", subcore_axis_name="s")

    nbuf = 4
    per_body = 8
    n_body = n_ch // per_body

    @functools.partial(
        pl.kernel, mesh=mesh,
        out_type=jax.ShapeDtypeStruct((B, _D), jnp.float32),
        scratch_types=[pltpu.VMEM((nbuf, _CH), jnp.int32),
                       pltpu.VMEM((nbuf, _CH, _D), jnp.float32),
                       pltpu.SemaphoreType.DMA((nbuf,)),
                       pltpu.SemaphoreType.DMA((nbuf,))],
    )
    def k(h_hbm, idx_hbm, out_hbm, idx_v, rows_v, gsems, osems):
        wid = lax.axis_index("s") * 2 + lax.axis_index("c")
        base = wid * per_w

        def body(t, carry):
            base_c = base + t * per_body * _CH
            gat = [None] * nbuf
            outs = [None] * nbuf
            for kk in range(per_body + nbuf - 1):
                bb = kk % nbuf
                if kk < per_body:
                    if kk >= nbuf:
                        outs[bb].wait()
                    off = base_c + kk * _CH
                    pltpu.sync_copy(idx_hbm.at[pl.ds(off, _CH)],
                                    idx_v.at[bb])
                    gat[bb] = pltpu.async_copy(
                        h_hbm.at[idx_v.at[bb]], rows_v.at[bb], gsems.at[bb])
                j = kk - (nbuf - 1)
                if j >= 0:
                    bj = j % nbuf
                    gat[bj].wait()
                    offj = base_c + j * _CH
                    outs[bj] = pltpu.async_copy(
                        rows_v.at[bj], out_hbm.at[pl.ds(offj, _CH)],
                        osems.at[bj])
            for bb in range(nbuf):
                outs[bb].wait()
            return carry

        lax.fori_loop(0, n_body, body, 0)

    return k(h, safe_flat)


_EP = 331776
_SB = 2048
_RPW = _NPAD // _NW
_NB1 = 80


def _select_topk(rows_p, cols_p):
    mesh = plsc.VectorSubcoreMesh(core_axis_name="c", subcore_axis_name="s")

    @functools.partial(
        pl.kernel, mesh=mesh,
        compiler_params=pltpu.CompilerParams(needs_layout_passes=False),
        out_type=jax.ShapeDtypeStruct((_NPAD * _K,), jnp.int32),
        scratch_types=[
            pltpu.VMEM((2, _SB), jnp.int32),
            pltpu.VMEM((2, _SB), jnp.int32),
            pltpu.SemaphoreType.DMA((2,)),
            pltpu.SemaphoreType.DMA((2,)),
            pltpu.VMEM((_RPW * _NB1,), jnp.int32),
            pltpu.VMEM((_RPW * 128,), jnp.int32),
            pltpu.VMEM((_RPW * _K,), jnp.int32),
            pltpu.VMEM((_RPW,), jnp.int32),
            pltpu.VMEM((_RPW,), jnp.int32),
            pltpu.VMEM((_RPW,), jnp.int32),
            pltpu.VMEM((_RPW,), jnp.int32),
            pltpu.VMEM((_RPW,), jnp.int32),
            pltpu.VMEM((_RPW,), jnp.int32),
        ])
    def sel(rows_hbm, cols_hbm, out_hbm, rbuf, cbuf, ssr, ssc, h1, h2, tkl,
            bstar, cstar, need1, need2, sctr, ectr):
        wid = lax.axis_index("s") * 2 + lax.axis_index("c")
        base = wid * _RPW
        i16 = lax.broadcasted_iota(jnp.int32, (16,), 0)
        zeros16 = jnp.zeros((16,), jnp.int32)
        ones16 = jnp.ones((16,), jnp.int32)

        def zero_fill(ref, nwords):
            def zf(i, c):
                ref[pl.ds(i * 16, 16)] = zeros16
                return c
            lax.fori_loop(0, nwords // 16, zf, 0)

        zero_fill(h1, _RPW * _NB1)
        zero_fill(h2, _RPW * 128)
        zero_fill(sctr, _RPW)
        zero_fill(ectr, _RPW)

        def mf(i, c):
            tkl[pl.ds(i * 16, 16)] = zeros16 - 1
            return c
        lax.fori_loop(0, _RPW * _K // 16, mf, 0)

        nstg = _EP // _SB

        def scan_edges(chunk_fn):
            def start_buf(bb, s):
                pltpu.async_copy(rows_hbm.at[pl.ds(s * _SB, _SB)],
                                 rbuf.at[bb], ssr.at[bb])
                pltpu.async_copy(cols_hbm.at[pl.ds(s * _SB, _SB)],
                                 cbuf.at[bb], ssc.at[bb])

            def wait_buf(bb):
                pltpu.make_async_copy(rows_hbm.at[pl.ds(0, _SB)],
                                      rbuf.at[bb], ssr.at[bb]).wait()
                pltpu.make_async_copy(cols_hbm.at[pl.ds(0, _SB)],
                                      cbuf.at[bb], ssc.at[bb]).wait()

            def process(bb):
                def chunk(i, cc):
                    row = rbuf[bb, pl.ds(i * 16, 16)]
                    col = cbuf[bb, pl.ds(i * 16, 16)]
                    m = (row >= base) & (row < base + _RPW)
                    chunk_fn(row - base, col, m)
                    return cc
                lax.fori_loop(0, _SB // 16, chunk, 0)

            start_buf(0, 0)

            def stage2(t, c):
                s0 = t * 2
                start_buf(1, s0 + 1)
                wait_buf(0)
                process(0)

                @pl.when(s0 + 2 < nstg)
                def _():
                    start_buf(0, s0 + 2)
                wait_buf(1)
                process(1)
                return c
            lax.fori_loop(0, nstg // 2, stage2, 0)

        def p1(r_loc, col, m):
            plsc.addupdate_scatter(h1, [r_loc * _NB1 + (col >> 7)], ones16,
                                   mask=m)
        scan_edges(p1)

        def l1(rg, c):
            r16 = rg * 16 + i16

            def bb(bkt, carry):
                cum, bst, below = carry
                v = plsc.load_gather(h1, [r16 * _NB1 + bkt])
                ncum = cum + v
                lt = (ncum < _K).astype(jnp.int32)
                return (ncum, bst + lt, below + v * lt)
            cum, bst, below = lax.fori_loop(
                0, _NB1, bb, (zeros16, zeros16, zeros16))
            plsc.store_scatter(bstar, [r16], bst)
            plsc.store_scatter(need1, [r16], _K - below)
            return c
        lax.fori_loop(0, _RPW // 16, l1, 0)

        def p2(r_loc, col, m):
            bst = plsc.load_gather(bstar, [r_loc], mask=m)
            m2 = m & ((col >> 7) == bst)
            plsc.addupdate_scatter(h2, [r_loc * 128 + (col & 127)], ones16,
                                   mask=m2)
        scan_edges(p2)

        def l2(rg, c):
            r16 = rg * 16 + i16
            n1 = plsc.load_gather(need1, [r16])

            def cb(cc, carry):
                cum, cst, below = carry
                v = plsc.load_gather(h2, [r16 * 128 + cc])
                ncum = cum + v
                lt = (ncum < n1).astype(jnp.int32)
                return (ncum, cst + lt, below + v * lt)
            cum, cst, below = lax.fori_loop(
                0, 128, cb, (zeros16, zeros16, zeros16))
            bst = plsc.load_gather(bstar, [r16])
            plsc.store_scatter(cstar, [r16], bst * 128 + cst)
            plsc.store_scatter(need2, [r16], n1 - below)
            return c
        lax.fori_loop(0, _RPW // 16, l2, 0)

        def p3(r_loc, col, m):
            cst = plsc.load_gather(cstar, [r_loc], mask=m)
            keep_lt = m & (col < cst)
            meq = m & (col == cst)
            oldeq = plsc.load_gather(ectr, [r_loc], mask=meq)
            occ_eq, _ = plsc.scan_count(r_loc, mask=meq)
            n2 = plsc.load_gather(need2, [r_loc], mask=meq)
            keep_eq = meq & ((oldeq + occ_eq - 1) < n2)
            plsc.addupdate_scatter(ectr, [r_loc], ones16, mask=meq)
            keep = keep_lt | keep_eq
            olds = plsc.load_gather(sctr, [r_loc], mask=keep)
            occ_s, _ = plsc.scan_count(r_loc, mask=keep)
            slot = olds + occ_s - 1
            keep = keep & (slot < _K)
            plsc.store_scatter(tkl, [r_loc * _K + slot], col, mask=keep)
            plsc.addupdate_scatter(sctr, [r_loc], ones16, mask=keep)
        scan_edges(p3)

        pltpu.sync_copy(tkl, out_hbm.at[pl.ds(wid * _RPW * _K, _RPW * _K)])

    return sel(rows_p, cols_p)


def _linear(feat, W):
    return pl.pallas_call(
        _linear_kernel,
        grid=(10,),
        in_specs=[pl.BlockSpec((1000, _D), lambda i: (i, 0)),
                  pl.BlockSpec((_D, _D), lambda i: (0, 0))],
        out_specs=pl.BlockSpec((1000, _D), lambda i: (i, 0)),
        out_shape=jax.ShapeDtypeStruct((_N, _D), jnp.float32),
    )(feat, W)


def _attention(gathered, vcol, vrow, b):
    npart = vcol.shape[0] // _K
    nb = npart // _BLK
    return pl.pallas_call(
        _attn_kernel,
        grid=(nb,),
        in_specs=[pl.BlockSpec((_BLK * _K, _D), lambda i: (i, 0)),
                  pl.BlockSpec((_BLK * _K, 1), lambda i: (i, 0)),
                  pl.BlockSpec((_BLK // _GRP, _GRP * _K), lambda i: (i, 0)),
                  pl.BlockSpec((1, _D), lambda i: (0, 0))],
        out_specs=pl.BlockSpec((_BLK, _D), lambda i: (i, 0)),
        out_shape=jax.ShapeDtypeStruct((npart, _D), jnp.float32),
    )(gathered, vcol, vrow, b.reshape(1, _D))


def kernel(feat, edge_index, W, b):
    n = _N
    e = edge_index.shape[1]
    loops = jnp.arange(n, dtype=edge_index.dtype)
    pad = _EP - e - n
    rows_p = jnp.concatenate(
        [edge_index[0], loops, jnp.full((pad,), _NPAD - 1, jnp.int32)])
    cols_p = jnp.concatenate(
        [edge_index[1], loops, jnp.zeros((pad,), jnp.int32)])
    h = _linear(feat, W)
    topk_p = _select_topk(rows_p, cols_p).reshape(_NPAD, _K)
    safe = jnp.clip(topk_p, 0, n - 1)
    valid = (topk_p != -1).astype(jnp.float32)
    hb = _NPAD // 2
    parts = []
    for c in range(2):
        sl = slice(c * hb, (c + 1) * hb)
        g = _gather_rows(h, safe[sl].reshape(-1))
        vc = valid[sl].reshape(hb * _K, 1)
        vr = valid[sl].reshape(hb // _GRP, _GRP * _K)
        parts.append(_attention(g, vc, vr, b))
    out = jnp.concatenate(parts, axis=0)
    return out[:n]
